# Initial kernel scaffold; baseline (speedup 1.0000x reference)
import jax
import jax.numpy as jnp
from jax.experimental import pallas as pl


def kernel(n_feat, edge_index, edge_type, ori_edge_ids, rel_table, W0, b0, bn_scale, bn_bias, bn_mean, bn_var, W1, b1):
    raise NotImplementedError("write your pallas kernel here")


# speedup vs baseline: 6.2495x; 6.2495x over previous
import functools
import jax
import jax.numpy as jnp
from jax import lax
from jax.experimental import pallas as pl
from jax.experimental.pallas import tpu as pltpu
from jax.experimental.pallas import tpu_sc as plsc

N = 10000
E = 320000
E_ORI = 160000
D = 128
HID = 64
NREL = 16
LAMDA = 0.5
THRESH = 0.01

NC, NS = 2, 16
NW = NC * NS
EPW = E // NW
CH = 80
NCH = EPW // CH

NSEG = 10240
SEGW = NSEG // NS

IPW = E_ORI // NW
ICH = 80
NICH = IPW // ICH
ITAIL = IPW - NICH * ICH

MSK = E // NS


def _sc_mesh():
    return plsc.VectorSubcoreMesh(core_axis_name="c", subcore_axis_name="s")


def _wid():
    return lax.axis_index("s") * NC + lax.axis_index("c")


def _mask_body(ids_hbm, m_hbm, idx, idxt, ones, onest, zbuf, mbuf, m_sh, sem):
    c = lax.axis_index("c")
    s = lax.axis_index("s")
    wid = s * NC + c

    def zfill(i, carry):
        zbuf[pl.ds(i * 16, 16)] = jnp.zeros((16,), jnp.float32)
        return carry

    lax.fori_loop(0, 800 // 16, zfill, 0)

    def ofill(i, carry):
        ones[pl.ds(i * 16, 16)] = jnp.ones((16,), jnp.float32)
        return carry

    lax.fori_loop(0, ICH // 16, ofill, 0)

    for i in range((ITAIL + 15) // 16):
        onest[pl.ds(min(i * 16, ITAIL - 16), 16)] = jnp.ones((16,), jnp.float32)

    def zcopy(i, carry):
        pltpu.sync_copy(zbuf, m_sh.at[pl.ds(s * MSK + i * 800, 800)])
        return carry

    lax.fori_loop(0, MSK // 800, zcopy, 0)
    plsc.subcore_barrier()

    base = wid * IPW

    def chunk(i, carry):
        pltpu.sync_copy(ids_hbm.at[pl.ds(base + i * ICH, ICH)], idx)
        pltpu.sync_copy(ones, m_sh.at[idx])
        return carry

    lax.fori_loop(0, NICH, chunk, 0)
    pltpu.sync_copy(ids_hbm.at[pl.ds(base + NICH * ICH, ITAIL)], idxt)
    pltpu.sync_copy(onest, m_sh.at[idxt])
    plsc.subcore_barrier()
    moff = pl.multiple_of(c * E + s * MSK, 8)

    def dump(i, carry):
        pltpu.sync_copy(m_sh.at[pl.ds(s * MSK + i * 4000, 4000)], mbuf)
        pltpu.sync_copy(mbuf, m_hbm.at[pl.ds(moff + i * 4000, 4000)])
        return carry

    lax.fori_loop(0, MSK // 4000, dump, 0)


def _ori_mask(ori_edge_ids):
    k = pl.kernel(
        _mask_body,
        out_type=jax.ShapeDtypeStruct((NC * E,), jnp.float32),
        mesh=_sc_mesh(),
        scratch_types=[
            pltpu.VMEM((ICH,), jnp.int32),
            pltpu.VMEM((ITAIL,), jnp.int32),
            pltpu.VMEM((ICH,), jnp.float32),
            pltpu.VMEM((ITAIL,), jnp.float32),
            pltpu.VMEM((800,), jnp.float32),
            pltpu.VMEM((4000,), jnp.float32),
            pltpu.VMEM_SHARED((E,), jnp.float32),
            pltpu.SemaphoreType.DMA,
        ],
    )
    return k(ori_edge_ids)


def _gather_feat_body(nf_hbm, src_hbm, dst_hbm, g_hbm,
                      idx_s, idx_d, rows_s, rows_d, sem):
    base = _wid() * EPW

    def chunk(i, carry):
        off = base + i * CH
        pltpu.sync_copy(src_hbm.at[pl.ds(off, CH)], idx_s)
        pltpu.sync_copy(dst_hbm.at[pl.ds(off, CH)], idx_d)
        cps = pltpu.async_copy(nf_hbm.at[idx_s], rows_s, sem)
        cpd = pltpu.async_copy(nf_hbm.at[idx_d], rows_d, sem)
        cps.wait()
        cpd.wait()

        def edge(e, c2):
            for dk in range(D // 16):
                sl = pl.ds(dk * 16, 16)
                a = rows_s[e, sl]
                b = rows_d[e, sl]
                rows_s[e, sl] = jnp.exp(-jnp.abs(a - b))
            return c2

        lax.fori_loop(0, CH, edge, 0)
        pltpu.sync_copy(rows_s, g_hbm.at[pl.ds(off, CH)])
        return carry

    lax.fori_loop(0, NCH, chunk, 0)


def _gather_feat(n_feat, src, dst):
    k = pl.kernel(
        _gather_feat_body,
        out_type=jax.ShapeDtypeStruct((E, D), jnp.float32),
        mesh=_sc_mesh(),
        scratch_types=[
            pltpu.VMEM((CH,), jnp.int32),
            pltpu.VMEM((CH,), jnp.int32),
            pltpu.VMEM((CH, D), jnp.float32),
            pltpu.VMEM((CH, D), jnp.float32),
            pltpu.SemaphoreType.DMA,
        ],
    )
    return k(n_feat, src, dst)


TB = 2000


def _mlp_body(g_ref, et_ref, m0_ref, m1_ref, w0_ref, rb_ref, w1_ref, b1_ref,
              w_ref, gmax_ref):
    g = g_ref[...]
    et = et_ref[0, 0, :]
    onehot = (et[:, None] == lax.broadcasted_iota(jnp.int32, (TB, NREL), 1)
              ).astype(jnp.float32)
    h = (jnp.dot(g, w0_ref[...], preferred_element_type=jnp.float32)
         + jnp.dot(onehot, rb_ref[...], preferred_element_type=jnp.float32))
    h = jnp.where(h > 0, h, 0.01 * h)
    w = jnp.dot(h, w1_ref[...], preferred_element_type=jnp.float32)
    w = w + b1_ref[0, 0]
    mm = jnp.minimum(m0_ref[...] + m1_ref[...], 1.0)
    w = w * (1.0 - LAMDA * mm) + LAMDA * mm
    w_ref[...] = w
    tm = jnp.max(w, axis=0, keepdims=True)

    @pl.when(pl.program_id(0) == 0)
    def _():
        gmax_ref[...] = tm

    @pl.when(pl.program_id(0) != 0)
    def _():
        gmax_ref[...] = jnp.maximum(gmax_ref[...], tm)


def _mlp(g, edge_type, m0, m1, w0p, rb, w1, b1):
    et3 = edge_type.reshape(E // TB, 1, TB)
    return pl.pallas_call(
        _mlp_body,
        grid=(E // TB,),
        in_specs=[
            pl.BlockSpec((TB, D), lambda i: (i, 0)),
            pl.BlockSpec((1, 1, TB), lambda i: (i, 0, 0)),
            pl.BlockSpec((TB, 1), lambda i: (i, 0)),
            pl.BlockSpec((TB, 1), lambda i: (i, 0)),
            pl.BlockSpec((D, HID), lambda i: (0, 0)),
            pl.BlockSpec((NREL, HID), lambda i: (0, 0)),
            pl.BlockSpec((HID, 1), lambda i: (0, 0)),
            pl.BlockSpec((1, 1), lambda i: (0, 0)),
        ],
        out_specs=[
            pl.BlockSpec((TB, 1), lambda i: (i, 0)),
            pl.BlockSpec((1, 1), lambda i: (0, 0)),
        ],
        out_shape=[
            jax.ShapeDtypeStruct((E, 1), jnp.float32),
            jax.ShapeDtypeStruct((1, 1), jnp.float32),
        ],
    )(g, et3, m0, m1, w0p, rb, w1, b1)


DCH = 80
NDCH = EPW // DCH


def _segsum_body(w_hbm, dst_hbm, gm_hbm, ew_hbm, segs_hbm,
                 idx, vals, gmv, zbuf, seg_sh, sem):
    c = lax.axis_index("c")
    s = lax.axis_index("s")
    wid = s * NC + c
    base = wid * EPW

    pltpu.sync_copy(gm_hbm, gmv)

    def zfill(i, carry):
        zbuf[pl.ds(i * 16, 16)] = jnp.zeros((16,), jnp.float32)
        return carry

    lax.fori_loop(0, SEGW // 16, zfill, 0)
    pltpu.sync_copy(zbuf, seg_sh.at[pl.ds(s * SEGW, SEGW)])
    plsc.subcore_barrier()

    gm = gmv[...]

    def chunk(i, carry):
        off = base + i * DCH
        pltpu.sync_copy(w_hbm.at[pl.ds(off, DCH)], vals)
        pltpu.sync_copy(dst_hbm.at[pl.ds(off, DCH)], idx)

        def sub(k, c2):
            sl = pl.ds(k * 16, 16)
            vals[sl] = jnp.exp(vals[sl] - gm)
            return c2

        lax.fori_loop(0, DCH // 16, sub, 0)
        pltpu.sync_copy(vals, ew_hbm.at[pl.ds(off, DCH)])
        pltpu.sync_copy(vals, seg_sh.at[idx], add=True)
        return carry

    lax.fori_loop(0, NDCH, chunk, 0)
    plsc.subcore_barrier()
    soff = pl.multiple_of(c * NSEG + s * SEGW, 8)
    pltpu.sync_copy(seg_sh.at[pl.ds(s * SEGW, SEGW)], zbuf)
    pltpu.sync_copy(zbuf, segs_hbm.at[pl.ds(soff, SEGW)])


def _segsum(w, dst, gm16):
    k = pl.kernel(
        _segsum_body,
        out_type=[
            jax.ShapeDtypeStruct((E,), jnp.float32),
            jax.ShapeDtypeStruct((NC * NSEG,), jnp.float32),
        ],
        mesh=_sc_mesh(),
        scratch_types=[
            pltpu.VMEM((DCH,), jnp.int32),
            pltpu.VMEM((DCH,), jnp.float32),
            pltpu.VMEM((16,), jnp.float32),
            pltpu.VMEM((SEGW,), jnp.float32),
            pltpu.VMEM_SHARED((NSEG,), jnp.float32),
            pltpu.SemaphoreType.DMA,
        ],
    )
    return k(w, dst, gm16)


ECH = 80
NECH = EPW // ECH


def _norm_body(ew_hbm, dst_hbm, segs_hbm, out_hbm,
               idx, idx2, vals, sv0, sv1, sem):
    base = _wid() * EPW

    def chunk(i, carry):
        off = base + i * ECH
        pltpu.sync_copy(ew_hbm.at[pl.ds(off, ECH)], vals)
        pltpu.sync_copy(dst_hbm.at[pl.ds(off, ECH)], idx)

        def mkidx(k, c2):
            sl = pl.ds(k * 16, 16)
            idx2[sl] = idx[sl] + NSEG
            return c2

        lax.fori_loop(0, ECH // 16, mkidx, 0)
        cp0 = pltpu.async_copy(segs_hbm.at[idx], sv0, sem)
        cp1 = pltpu.async_copy(segs_hbm.at[idx2], sv1, sem)
        cp0.wait()
        cp1.wait()

        def sub(k, c2):
            sl = pl.ds(k * 16, 16)
            r = vals[sl] / (sv0[sl] + sv1[sl])
            vals[sl] = jnp.where(r > THRESH, r, 0.0)
            return c2

        lax.fori_loop(0, ECH // 16, sub, 0)
        pltpu.sync_copy(vals, out_hbm.at[pl.ds(off, ECH)])
        return carry

    lax.fori_loop(0, NECH, chunk, 0)


def _normalize(ew, dst, segs):
    k = pl.kernel(
        _norm_body,
        out_type=jax.ShapeDtypeStruct((E,), jnp.float32),
        mesh=_sc_mesh(),
        scratch_types=[
            pltpu.VMEM((ECH,), jnp.int32),
            pltpu.VMEM((ECH,), jnp.int32),
            pltpu.VMEM((ECH,), jnp.float32),
            pltpu.VMEM((ECH,), jnp.float32),
            pltpu.VMEM((ECH,), jnp.float32),
            pltpu.SemaphoreType.DMA,
        ],
    )
    return k(ew, dst, segs)


def kernel(n_feat, edge_index, edge_type, ori_edge_ids, rel_table,
           W0, b0, bn_scale, bn_bias, bn_mean, bn_var, W1, b1):
    src = edge_index[0]
    dst = edge_index[1]

    s = bn_scale / jnp.sqrt(bn_var + 1e-5)
    t = bn_bias - bn_mean * s
    w0p = W0[:D] * s[None, :]
    rb = rel_table @ (W0[D:] * s[None, :]) + (b0 * s + t)[None, :]

    m = _ori_mask(ori_edge_ids)
    g = _gather_feat(n_feat, src, dst)
    w, gmax = _mlp(g, edge_type, m[:E][:, None], m[E:][:, None],
                   w0p, rb, W1, b1.reshape(1, 1))
    gm16 = jnp.broadcast_to(gmax.reshape(1), (16,))
    ew, segs = _segsum(w[:, 0], dst, gm16)
    out = _normalize(ew, dst, segs)
    return out[:, None]


# speedup vs baseline: 7.6374x; 1.2221x over previous
import functools
import jax
import jax.numpy as jnp
from jax import lax
from jax.experimental import pallas as pl
from jax.experimental.pallas import tpu as pltpu
from jax.experimental.pallas import tpu_sc as plsc

N = 10000
E = 320000
E_ORI = 160000
D = 128
HID = 64
NREL = 16
LAMDA = 0.5
THRESH = 0.01

NC, NS = 2, 16
NW = NC * NS
EPW = E // NW
CH = 80
NCH = EPW // CH

NSEG = 10240
SEGW = NSEG // NS

IPW = E_ORI // NW
ICH = 80
NICH = IPW // ICH
ITAIL = IPW - NICH * ICH

MSK = E // NS


def _sc_mesh():
    return plsc.VectorSubcoreMesh(core_axis_name="c", subcore_axis_name="s")


def _wid():
    return lax.axis_index("s") * NC + lax.axis_index("c")


def _mask_body(ids_hbm, m_hbm, idx, idxt, ones, onest, zbuf, mbuf, m_sh, sem):
    c = lax.axis_index("c")
    s = lax.axis_index("s")
    wid = s * NC + c

    def zfill(i, carry):
        zbuf[pl.ds(i * 16, 16)] = jnp.zeros((16,), jnp.float32)
        return carry

    lax.fori_loop(0, 800 // 16, zfill, 0)

    def ofill(i, carry):
        ones[pl.ds(i * 16, 16)] = jnp.ones((16,), jnp.float32)
        return carry

    lax.fori_loop(0, ICH // 16, ofill, 0)

    for i in range((ITAIL + 15) // 16):
        onest[pl.ds(min(i * 16, ITAIL - 16), 16)] = jnp.ones((16,), jnp.float32)

    def zcopy(i, carry):
        pltpu.sync_copy(zbuf, m_sh.at[pl.ds(s * MSK + i * 800, 800)])
        return carry

    lax.fori_loop(0, MSK // 800, zcopy, 0)
    plsc.subcore_barrier()

    base = wid * IPW

    def chunk(i, carry):
        pltpu.sync_copy(ids_hbm.at[pl.ds(base + i * ICH, ICH)], idx)
        pltpu.sync_copy(ones, m_sh.at[idx])
        return carry

    lax.fori_loop(0, NICH, chunk, 0)
    pltpu.sync_copy(ids_hbm.at[pl.ds(base + NICH * ICH, ITAIL)], idxt)
    pltpu.sync_copy(onest, m_sh.at[idxt])
    plsc.subcore_barrier()
    moff = pl.multiple_of(c * E + s * MSK, 8)

    def dump(i, carry):
        pltpu.sync_copy(m_sh.at[pl.ds(s * MSK + i * 4000, 4000)], mbuf)
        pltpu.sync_copy(mbuf, m_hbm.at[pl.ds(moff + i * 4000, 4000)])
        return carry

    lax.fori_loop(0, MSK // 4000, dump, 0)


def _ori_mask(ori_edge_ids):
    k = pl.kernel(
        _mask_body,
        out_type=jax.ShapeDtypeStruct((NC * E,), jnp.float32),
        mesh=_sc_mesh(),
        scratch_types=[
            pltpu.VMEM((ICH,), jnp.int32),
            pltpu.VMEM((ITAIL,), jnp.int32),
            pltpu.VMEM((ICH,), jnp.float32),
            pltpu.VMEM((ITAIL,), jnp.float32),
            pltpu.VMEM((800,), jnp.float32),
            pltpu.VMEM((4000,), jnp.float32),
            pltpu.VMEM_SHARED((E,), jnp.float32),
            pltpu.SemaphoreType.DMA,
        ],
    )
    return k(ori_edge_ids)


def _gather_feat_body(nf_hbm, src_hbm, dst_hbm, g_hbm,
                      idx_s0, idx_d0, idx_s1, idx_d1,
                      rows_s0, rows_d0, rows_s1, rows_d1,
                      obuf0, obuf1, gsem0, gsem1, wsem0, wsem1):
    base = _wid() * EPW
    idx_s = (idx_s0, idx_s1)
    idx_d = (idx_d0, idx_d1)
    rows_s = (rows_s0, rows_s1)
    rows_d = (rows_d0, rows_d1)
    obuf = (obuf0, obuf1)
    gsem = (gsem0, gsem1)
    wsem = (wsem0, wsem1)

    def fire(i, b):
        off = pl.multiple_of(base + i * CH, 8)
        pltpu.sync_copy(src_hbm.at[pl.ds(off, CH)], idx_s[b])
        pltpu.sync_copy(dst_hbm.at[pl.ds(off, CH)], idx_d[b])
        pltpu.async_copy(nf_hbm.at[idx_s[b]], rows_s[b], gsem[b])
        pltpu.async_copy(nf_hbm.at[idx_d[b]], rows_d[b], gsem[b])

    def drain_write(b):
        pltpu.make_async_copy(obuf[b], g_hbm.at[pl.ds(base, CH)], wsem[b]).wait()

    def process(i, b):
        pltpu.make_async_copy(nf_hbm.at[idx_s[b]], rows_s[b], gsem[b]).wait()
        pltpu.make_async_copy(nf_hbm.at[idx_d[b]], rows_d[b], gsem[b]).wait()

        def edge(e, c2):
            for dk in range(D // 16):
                sl = pl.ds(dk * 16, 16)
                a = rows_s[b][e, sl]
                bb = rows_d[b][e, sl]
                obuf[b][e, sl] = jnp.exp(-jnp.abs(a - bb))
            return c2

        lax.fori_loop(0, CH, edge, 0)
        off = pl.multiple_of(base + i * CH, 8)
        pltpu.async_copy(obuf[b], g_hbm.at[pl.ds(off, CH)], wsem[b])

    fire(0, 0)

    def body(j, carry):
        fire(2 * j + 1, 1)

        @pl.when(j > 0)
        def _():
            drain_write(0)

        process(2 * j, 0)
        fire(2 * j + 2, 0)

        @pl.when(j > 0)
        def _():
            drain_write(1)

        process(2 * j + 1, 1)
        return carry

    lax.fori_loop(0, (NCH - 1) // 2, body, 0)
    drain_write(0)
    process(NCH - 1, 0)
    drain_write(1)
    drain_write(0)


def _gather_feat(n_feat, src, dst):
    k = pl.kernel(
        _gather_feat_body,
        out_type=jax.ShapeDtypeStruct((E, D), jnp.float32),
        mesh=_sc_mesh(),
        scratch_types=[
            pltpu.VMEM((CH,), jnp.int32),
            pltpu.VMEM((CH,), jnp.int32),
            pltpu.VMEM((CH,), jnp.int32),
            pltpu.VMEM((CH,), jnp.int32),
            pltpu.VMEM((CH, D), jnp.float32),
            pltpu.VMEM((CH, D), jnp.float32),
            pltpu.VMEM((CH, D), jnp.float32),
            pltpu.VMEM((CH, D), jnp.float32),
            pltpu.VMEM((CH, D), jnp.float32),
            pltpu.VMEM((CH, D), jnp.float32),
            pltpu.SemaphoreType.DMA,
            pltpu.SemaphoreType.DMA,
            pltpu.SemaphoreType.DMA,
            pltpu.SemaphoreType.DMA,
        ],
    )
    return k(n_feat, src, dst)


TB = 2000


def _mlp_body(g_ref, et_ref, m0_ref, m1_ref, w0_ref, rb_ref, w1_ref, b1_ref,
              w_ref, gmax_ref):
    g = g_ref[...]
    et = et_ref[0, 0, :]
    onehot = (et[:, None] == lax.broadcasted_iota(jnp.int32, (TB, NREL), 1)
              ).astype(jnp.float32)
    h = (jnp.dot(g, w0_ref[...], preferred_element_type=jnp.float32)
         + jnp.dot(onehot, rb_ref[...], preferred_element_type=jnp.float32))
    h = jnp.where(h > 0, h, 0.01 * h)
    w = jnp.dot(h, w1_ref[...], preferred_element_type=jnp.float32)
    w = w + b1_ref[0, 0]
    mm = jnp.minimum(m0_ref[...] + m1_ref[...], 1.0)
    w = w * (1.0 - LAMDA * mm) + LAMDA * mm
    w_ref[...] = w
    tm = jnp.max(w, axis=0, keepdims=True)

    @pl.when(pl.program_id(0) == 0)
    def _():
        gmax_ref[...] = tm

    @pl.when(pl.program_id(0) != 0)
    def _():
        gmax_ref[...] = jnp.maximum(gmax_ref[...], tm)


def _mlp(g, edge_type, m0, m1, w0p, rb, w1, b1):
    et3 = edge_type.reshape(E // TB, 1, TB)
    return pl.pallas_call(
        _mlp_body,
        grid=(E // TB,),
        in_specs=[
            pl.BlockSpec((TB, D), lambda i: (i, 0)),
            pl.BlockSpec((1, 1, TB), lambda i: (i, 0, 0)),
            pl.BlockSpec((TB, 1), lambda i: (i, 0)),
            pl.BlockSpec((TB, 1), lambda i: (i, 0)),
            pl.BlockSpec((D, HID), lambda i: (0, 0)),
            pl.BlockSpec((NREL, HID), lambda i: (0, 0)),
            pl.BlockSpec((HID, 1), lambda i: (0, 0)),
            pl.BlockSpec((1, 1), lambda i: (0, 0)),
        ],
        out_specs=[
            pl.BlockSpec((TB, 1), lambda i: (i, 0)),
            pl.BlockSpec((1, 1), lambda i: (0, 0)),
        ],
        out_shape=[
            jax.ShapeDtypeStruct((E, 1), jnp.float32),
            jax.ShapeDtypeStruct((1, 1), jnp.float32),
        ],
    )(g, et3, m0, m1, w0p, rb, w1, b1)


DCH = 80
NDCH = EPW // DCH


def _segsum_body(w_hbm, dst_hbm, gm_hbm, ew_hbm, segs_hbm,
                 idx, vals, gmv, zbuf, seg_sh, sem):
    c = lax.axis_index("c")
    s = lax.axis_index("s")
    wid = s * NC + c
    base = wid * EPW

    pltpu.sync_copy(gm_hbm, gmv)

    def zfill(i, carry):
        zbuf[pl.ds(i * 16, 16)] = jnp.zeros((16,), jnp.float32)
        return carry

    lax.fori_loop(0, SEGW // 16, zfill, 0)
    pltpu.sync_copy(zbuf, seg_sh.at[pl.ds(s * SEGW, SEGW)])
    plsc.subcore_barrier()

    gm = gmv[...]

    def chunk(i, carry):
        off = base + i * DCH
        pltpu.sync_copy(w_hbm.at[pl.ds(off, DCH)], vals)
        pltpu.sync_copy(dst_hbm.at[pl.ds(off, DCH)], idx)

        def sub(k, c2):
            sl = pl.ds(k * 16, 16)
            vals[sl] = jnp.exp(vals[sl] - gm)
            return c2

        lax.fori_loop(0, DCH // 16, sub, 0)
        pltpu.sync_copy(vals, ew_hbm.at[pl.ds(off, DCH)])
        pltpu.sync_copy(vals, seg_sh.at[idx], add=True)
        return carry

    lax.fori_loop(0, NDCH, chunk, 0)
    plsc.subcore_barrier()
    soff = pl.multiple_of(c * NSEG + s * SEGW, 8)
    pltpu.sync_copy(seg_sh.at[pl.ds(s * SEGW, SEGW)], zbuf)
    pltpu.sync_copy(zbuf, segs_hbm.at[pl.ds(soff, SEGW)])


def _segsum(w, dst, gm16):
    k = pl.kernel(
        _segsum_body,
        out_type=[
            jax.ShapeDtypeStruct((E,), jnp.float32),
            jax.ShapeDtypeStruct((NC * NSEG,), jnp.float32),
        ],
        mesh=_sc_mesh(),
        scratch_types=[
            pltpu.VMEM((DCH,), jnp.int32),
            pltpu.VMEM((DCH,), jnp.float32),
            pltpu.VMEM((16,), jnp.float32),
            pltpu.VMEM((SEGW,), jnp.float32),
            pltpu.VMEM_SHARED((NSEG,), jnp.float32),
            pltpu.SemaphoreType.DMA,
        ],
    )
    return k(w, dst, gm16)


ECH = 80
NECH = EPW // ECH


def _norm_body(ew_hbm, dst_hbm, segs_hbm, out_hbm,
               idx, idx2, vals, sv0, sv1, sem):
    base = _wid() * EPW

    def chunk(i, carry):
        off = base + i * ECH
        pltpu.sync_copy(ew_hbm.at[pl.ds(off, ECH)], vals)
        pltpu.sync_copy(dst_hbm.at[pl.ds(off, ECH)], idx)

        def mkidx(k, c2):
            sl = pl.ds(k * 16, 16)
            idx2[sl] = idx[sl] + NSEG
            return c2

        lax.fori_loop(0, ECH // 16, mkidx, 0)
        cp0 = pltpu.async_copy(segs_hbm.at[idx], sv0, sem)
        cp1 = pltpu.async_copy(segs_hbm.at[idx2], sv1, sem)
        cp0.wait()
        cp1.wait()

        def sub(k, c2):
            sl = pl.ds(k * 16, 16)
            r = vals[sl] / (sv0[sl] + sv1[sl])
            vals[sl] = jnp.where(r > THRESH, r, 0.0)
            return c2

        lax.fori_loop(0, ECH // 16, sub, 0)
        pltpu.sync_copy(vals, out_hbm.at[pl.ds(off, ECH)])
        return carry

    lax.fori_loop(0, NECH, chunk, 0)


def _normalize(ew, dst, segs):
    k = pl.kernel(
        _norm_body,
        out_type=jax.ShapeDtypeStruct((E,), jnp.float32),
        mesh=_sc_mesh(),
        scratch_types=[
            pltpu.VMEM((ECH,), jnp.int32),
            pltpu.VMEM((ECH,), jnp.int32),
            pltpu.VMEM((ECH,), jnp.float32),
            pltpu.VMEM((ECH,), jnp.float32),
            pltpu.VMEM((ECH,), jnp.float32),
            pltpu.SemaphoreType.DMA,
        ],
    )
    return k(ew, dst, segs)


def kernel(n_feat, edge_index, edge_type, ori_edge_ids, rel_table,
           W0, b0, bn_scale, bn_bias, bn_mean, bn_var, W1, b1):
    src = edge_index[0]
    dst = edge_index[1]

    s = bn_scale / jnp.sqrt(bn_var + 1e-5)
    t = bn_bias - bn_mean * s
    w0p = W0[:D] * s[None, :]
    rb = rel_table @ (W0[D:] * s[None, :]) + (b0 * s + t)[None, :]

    m = _ori_mask(ori_edge_ids)
    g = _gather_feat(n_feat, src, dst)
    w, gmax = _mlp(g, edge_type, m[:E][:, None], m[E:][:, None],
                   w0p, rb, W1, b1.reshape(1, 1))
    gm16 = jnp.broadcast_to(gmax.reshape(1), (16,))
    ew, segs = _segsum(w[:, 0], dst, gm16)
    out = _normalize(ew, dst, segs)
    return out[:, None]
